# baseline (speedup 1.0000x reference)
import functools

import jax
import jax.numpy as jnp
import numpy as np
from jax import lax
from jax.experimental import pallas as pl
from jax.experimental.pallas import tpu as pltpu
from jax.experimental.pallas import tpu_sc as plsc

_NUM_CLUSTERS = 100000
_DIM = 64
_NUM_LABELS = 128
_PAD_ID = _NUM_CLUSTERS
_B = 4096
_L = 200

_NC = 2
_NS = 16
_NW = _NC * _NS
_ROWS_PER_W = _B // _NW
_LANES = 16

_VROWS = 100352
_WPR = 40
_CL_PER_W = _VROWS // _NW
_CHUNK = 448
_COLS_PAD = 100096
_LAST_START = _COLS_PAD - _CHUNK

_PERM = np.concatenate([
    np.arange(0, 32, 2), np.arange(1, 32, 2),
    np.arange(32, 64, 2), np.arange(33, 64, 2),
])

_MESH = dict(core_axis_name="c", subcore_axis_name="s")
_CPARAMS = pltpu.CompilerParams(
    use_tc_tiling_on_sc=False, needs_layout_passes=False)


def _sc_pack_table(embT):
    @functools.partial(
        pl.kernel,
        mesh=plsc.VectorSubcoreMesh(**_MESH),
        out_type=jax.ShapeDtypeStruct((_VROWS, _WPR), jnp.int32),
        compiler_params=_CPARAMS,
        scratch_types=[
            pltpu.VMEM((2, _DIM, _CHUNK), jnp.float32),
            pltpu.VMEM((_CHUNK, _WPR), jnp.int32),
            pltpu.SemaphoreType.DMA((2,)),
        ],
    )
    def k(embT_hbm, out_hbm, in_v, out_v, sem):
        wid = lax.axis_index("s") * _NC + lax.axis_index("c")
        base = wid * _CL_PER_W
        himask = jnp.full((_LANES,), -65536, jnp.int32)
        nch = _CL_PER_W // _CHUNK

        def incp(ch, par):
            c0 = jnp.minimum(base + ch * _CHUNK, _LAST_START)
            return pltpu.make_async_copy(
                embT_hbm.at[:, pl.ds(c0, _CHUNK)], in_v.at[par],
                sem.at[par])

        incp(0, 0).start()

        def chunk(ch, carry):
            par = ch & 1

            @pl.when(ch < nch - 1)
            def _():
                incp(ch + 1, 1 - par).start()

            incp(ch, par).wait()

            def grp(g, carry2):
                rows = lax.iota(jnp.int32, _LANES) + g * _LANES
                for t in range(32):
                    a = plsc.bitcast(
                        in_v[par, 2 * t, pl.ds(g * _LANES, _LANES)],
                        jnp.int32)
                    b = plsc.bitcast(
                        in_v[par, 2 * t + 1, pl.ds(g * _LANES, _LANES)],
                        jnp.int32)
                    w = lax.bitwise_or(
                        lax.shift_right_logical(a, 16),
                        lax.bitwise_and(b, himask))
                    plsc.store_scatter(
                        out_v, [rows, jnp.full((_LANES,), t, jnp.int32)], w)
                return carry2

            lax.fori_loop(0, _CHUNK // _LANES, grp, 0)
            c0 = jnp.minimum(base + ch * _CHUNK, _LAST_START)
            pltpu.sync_copy(out_v, out_hbm.at[pl.ds(c0, _CHUNK)])
            return carry

        lax.fori_loop(0, nch, chunk, 0)

    return k(embT)


def _sc_gather_sum(ids, packed):
    @functools.partial(
        pl.kernel,
        mesh=plsc.VectorSubcoreMesh(**_MESH),
        out_type=jax.ShapeDtypeStruct((_B, _DIM), jnp.float32),
        compiler_params=_CPARAMS,
        scratch_types=[
            pltpu.VMEM((_ROWS_PER_W, _L), jnp.int32),
            pltpu.VMEM((2, _L, _WPR), jnp.int32),
            pltpu.VMEM((_ROWS_PER_W, _DIM), jnp.float32),
            pltpu.SemaphoreType.DMA((2,)),
        ],
    )
    def k(ids_hbm, emb_hbm, out_hbm, idx_v, buf_v, acc_v, sem):
        wid = lax.axis_index("s") * _NC + lax.axis_index("c")
        base = wid * _ROWS_PER_W
        pltpu.sync_copy(ids_hbm.at[pl.ds(base, _ROWS_PER_W)], idx_v)

        def copies(r, par):
            return (
                pltpu.make_async_copy(
                    emb_hbm.at[idx_v.at[r, pl.ds(0, 128)]],
                    buf_v.at[par, pl.ds(0, 128)], sem.at[par]),
                pltpu.make_async_copy(
                    emb_hbm.at[idx_v.at[r, pl.ds(128, _L - 128)]],
                    buf_v.at[par, pl.ds(128, _L - 128)], sem.at[par]),
            )

        def fire(r, par):
            for cp in copies(r, par):
                cp.start()

        def drain(r, par):
            for cp in copies(r, par):
                cp.wait()

        fire(0, 0)
        himask = jnp.full((_LANES,), -65536, jnp.int32)

        def row(r, carry):
            par = r & 1

            @pl.when(r < _ROWS_PER_W - 1)
            def _():
                fire(r + 1, 1 - par)

            drain(r, par)

            def red(j, accs):
                a0, a1, a2, a3 = accs
                for u in range(4):
                    for c in range(2):
                        x = buf_v[par, j * 4 + u,
                                  pl.ds(c * _LANES, _LANES)]
                        lo = plsc.bitcast(lax.shift_left(x, 16), jnp.float32)
                        hi = plsc.bitcast(lax.bitwise_and(x, himask),
                                          jnp.float32)
                        if c == 0:
                            a0 = a0 + lo
                            a1 = a1 + hi
                        else:
                            a2 = a2 + lo
                            a3 = a3 + hi
                return (a0, a1, a2, a3)

            zeros = tuple(
                jnp.zeros((_LANES,), jnp.float32) for _ in range(4))
            accs = lax.fori_loop(0, _L // 4, red, zeros)
            for c in range(4):
                acc_v[r, pl.ds(c * _LANES, _LANES)] = accs[c]
            return carry

        lax.fori_loop(0, _ROWS_PER_W, row, 0)
        pltpu.sync_copy(acc_v, out_hbm.at[pl.ds(base, _ROWS_PER_W)])

    return k(ids, packed)


def _tc_head(input_ids, emb_sum, Wp, b2d):
    def body(ids_ref, es_ref, w_ref, b_ref, out_ref):
        ids = ids_ref[...]
        cnt = jnp.sum((ids != _PAD_ID).astype(jnp.float32), axis=1,
                      keepdims=True)
        pooled = es_ref[...] / jnp.maximum(cnt, 1.0)
        out_ref[...] = (
            jnp.dot(pooled, w_ref[...], preferred_element_type=jnp.float32)
            + b_ref[...])

    return pl.pallas_call(
        body,
        out_shape=jax.ShapeDtypeStruct((_B, _NUM_LABELS), jnp.float32),
    )(input_ids, emb_sum, Wp, b2d)


def kernel(input_ids, embedding, W, b):
    ids = input_ids.astype(jnp.int32)
    embTp = jnp.pad(embedding.T, ((0, 0), (0, _COLS_PAD - _NUM_CLUSTERS - 1)))
    packed = _sc_pack_table(embTp)
    emb_sum = _sc_gather_sum(ids, packed)
    Wp = W[jnp.asarray(_PERM), :]
    return _tc_head(ids, emb_sum, Wp, b.reshape(1, _NUM_LABELS))


# speedup vs baseline: 1.0177x; 1.0177x over previous
import functools

import jax
import jax.numpy as jnp
import numpy as np
from jax import lax
from jax.experimental import pallas as pl
from jax.experimental.pallas import tpu as pltpu
from jax.experimental.pallas import tpu_sc as plsc

_NUM_CLUSTERS = 100000
_DIM = 64
_NUM_LABELS = 128
_PAD_ID = _NUM_CLUSTERS
_B = 4096
_L = 200

_NC = 2
_NS = 16
_NW = _NC * _NS
_ROWS_PER_W = _B // _NW
_LANES = 16

_VROWS = 100352
_WPR = 40
_CL_PER_W = _VROWS // _NW
_CHUNK = 448
_COLS_PAD = 100096
_LAST_START = _COLS_PAD - _CHUNK

_PERM = np.concatenate([
    np.arange(0, 32, 2), np.arange(1, 32, 2),
    np.arange(32, 64, 2), np.arange(33, 64, 2),
])

_MESH = dict(core_axis_name="c", subcore_axis_name="s")
_CPARAMS = pltpu.CompilerParams(
    use_tc_tiling_on_sc=False, needs_layout_passes=False)


def _sc_pack_table(embT):
    @functools.partial(
        pl.kernel,
        mesh=plsc.VectorSubcoreMesh(**_MESH),
        out_type=jax.ShapeDtypeStruct((_VROWS, 32), jnp.int32),
        compiler_params=_CPARAMS,
        scratch_types=[
            pltpu.VMEM((2, _DIM, _CHUNK), jnp.float32),
            pltpu.VMEM((_CHUNK, _WPR), jnp.int32),
            pltpu.SemaphoreType.DMA((2,)),
        ],
    )
    def k(embT_hbm, out_hbm, in_v, out_v, sem):
        wid = lax.axis_index("s") * _NC + lax.axis_index("c")
        base = wid * _CL_PER_W
        himask = jnp.full((_LANES,), -65536, jnp.int32)
        nch = _CL_PER_W // _CHUNK

        def incp(ch, par):
            c0 = jnp.minimum(base + ch * _CHUNK, _LAST_START)
            return pltpu.make_async_copy(
                embT_hbm.at[:, pl.ds(c0, _CHUNK)], in_v.at[par],
                sem.at[par])

        incp(0, 0).start()

        def chunk(ch, carry):
            par = ch & 1

            @pl.when(ch < nch - 1)
            def _():
                incp(ch + 1, 1 - par).start()

            incp(ch, par).wait()

            def grp(g, carry2):
                rows = lax.iota(jnp.int32, _LANES) + g * _LANES
                for t in range(32):
                    a = plsc.bitcast(
                        in_v[par, 2 * t, pl.ds(g * _LANES, _LANES)],
                        jnp.int32)
                    b = plsc.bitcast(
                        in_v[par, 2 * t + 1, pl.ds(g * _LANES, _LANES)],
                        jnp.int32)
                    w = lax.bitwise_or(
                        lax.shift_right_logical(a, 16),
                        lax.bitwise_and(b, himask))
                    plsc.store_scatter(
                        out_v, [rows, jnp.full((_LANES,), t, jnp.int32)], w)
                return carry2

            lax.fori_loop(0, _CHUNK // _LANES, grp, 0)
            c0 = jnp.minimum(base + ch * _CHUNK, _LAST_START)
            pltpu.sync_copy(out_v.at[:, pl.ds(0, 32)],
                            out_hbm.at[pl.ds(c0, _CHUNK)])
            return carry

        lax.fori_loop(0, nch, chunk, 0)

    return k(embT)


def _sc_gather_sum(ids, packed):
    @functools.partial(
        pl.kernel,
        mesh=plsc.VectorSubcoreMesh(**_MESH),
        out_type=jax.ShapeDtypeStruct((_B, _DIM), jnp.float32),
        compiler_params=_CPARAMS,
        scratch_types=[
            pltpu.VMEM((_ROWS_PER_W, _L), jnp.int32),
            pltpu.VMEM((2, _L, 32), jnp.int32),
            pltpu.VMEM((_ROWS_PER_W, _DIM), jnp.float32),
            pltpu.SemaphoreType.DMA((2,)),
        ],
    )
    def k(ids_hbm, emb_hbm, out_hbm, idx_v, buf_v, acc_v, sem):
        wid = lax.axis_index("s") * _NC + lax.axis_index("c")
        base = wid * _ROWS_PER_W
        pltpu.sync_copy(ids_hbm.at[pl.ds(base, _ROWS_PER_W)], idx_v)

        def copies(r, par):
            return (
                pltpu.make_async_copy(
                    emb_hbm.at[idx_v.at[r, pl.ds(0, 128)]],
                    buf_v.at[par, pl.ds(0, 128)], sem.at[par]),
                pltpu.make_async_copy(
                    emb_hbm.at[idx_v.at[r, pl.ds(128, _L - 128)]],
                    buf_v.at[par, pl.ds(128, _L - 128)], sem.at[par]),
            )

        def fire(r, par):
            for cp in copies(r, par):
                cp.start()

        def drain(r, par):
            for cp in copies(r, par):
                cp.wait()

        fire(0, 0)
        himask = jnp.full((_LANES,), -65536, jnp.int32)

        def row(r, carry):
            par = r & 1

            @pl.when(r < _ROWS_PER_W - 1)
            def _():
                fire(r + 1, 1 - par)

            drain(r, par)

            def red(j, accs):
                a0, a1, a2, a3 = accs
                for u in range(4):
                    for c in range(2):
                        x = buf_v[par, j * 4 + u,
                                  pl.ds(c * _LANES, _LANES)]
                        lo = plsc.bitcast(lax.shift_left(x, 16), jnp.float32)
                        hi = plsc.bitcast(lax.bitwise_and(x, himask),
                                          jnp.float32)
                        if c == 0:
                            a0 = a0 + lo
                            a1 = a1 + hi
                        else:
                            a2 = a2 + lo
                            a3 = a3 + hi
                return (a0, a1, a2, a3)

            zeros = tuple(
                jnp.zeros((_LANES,), jnp.float32) for _ in range(4))
            accs = lax.fori_loop(0, _L // 4, red, zeros)
            for c in range(4):
                acc_v[r, pl.ds(c * _LANES, _LANES)] = accs[c]
            return carry

        lax.fori_loop(0, _ROWS_PER_W, row, 0)
        pltpu.sync_copy(acc_v, out_hbm.at[pl.ds(base, _ROWS_PER_W)])

    return k(ids, packed)


def _tc_head(input_ids, emb_sum, Wp, b2d):
    def body(ids_ref, es_ref, w_ref, b_ref, out_ref):
        ids = ids_ref[...]
        cnt = jnp.sum((ids != _PAD_ID).astype(jnp.float32), axis=1,
                      keepdims=True)
        pooled = es_ref[...] / jnp.maximum(cnt, 1.0)
        out_ref[...] = (
            jnp.dot(pooled, w_ref[...], preferred_element_type=jnp.float32)
            + b_ref[...])

    return pl.pallas_call(
        body,
        out_shape=jax.ShapeDtypeStruct((_B, _NUM_LABELS), jnp.float32),
    )(input_ids, emb_sum, Wp, b2d)


def kernel(input_ids, embedding, W, b):
    ids = input_ids.astype(jnp.int32)
    embTp = jnp.pad(embedding.T, ((0, 0), (0, _COLS_PAD - _NUM_CLUSTERS - 1)))
    packed = _sc_pack_table(embTp)
    emb_sum = _sc_gather_sum(ids, packed)
    Wp = W[jnp.asarray(_PERM), :]
    return _tc_head(ids, emb_sum, Wp, b.reshape(1, _NUM_LABELS))


# speedup vs baseline: 1.2250x; 1.2038x over previous
import functools

import jax
import jax.numpy as jnp
import numpy as np
from jax import lax
from jax.experimental import pallas as pl
from jax.experimental.pallas import tpu as pltpu
from jax.experimental.pallas import tpu_sc as plsc

_NUM_CLUSTERS = 100000
_DIM = 64
_NUM_LABELS = 128
_PAD_ID = _NUM_CLUSTERS
_B = 4096
_L = 200

_NC = 2
_NS = 16
_NW = _NC * _NS
_ROWS_PER_W = _B // _NW
_LANES = 16

_VROWS = 100352
_WPR = 40
_CL_PER_W = _VROWS // _NW
_CHUNK = 448
_COLS_PAD = 100096
_LAST_START = _COLS_PAD - _CHUNK

_PERM = np.concatenate([
    np.arange(0, 32, 2), np.arange(1, 32, 2),
    np.arange(32, 64, 2), np.arange(33, 64, 2),
])

_MESH = dict(core_axis_name="c", subcore_axis_name="s")
_CPARAMS = pltpu.CompilerParams(
    use_tc_tiling_on_sc=False, needs_layout_passes=False)


def _sc_pack_table(embT):
    @functools.partial(
        pl.kernel,
        mesh=plsc.VectorSubcoreMesh(**_MESH),
        out_type=jax.ShapeDtypeStruct((_VROWS, 32), jnp.int32),
        compiler_params=_CPARAMS,
        scratch_types=[
            pltpu.VMEM((2, _DIM, _CHUNK), jnp.float32),
            pltpu.VMEM((_CHUNK, _WPR), jnp.int32),
            pltpu.SemaphoreType.DMA((2,)),
        ],
    )
    def k(embT_hbm, out_hbm, in_v, out_v, sem):
        wid = lax.axis_index("s") * _NC + lax.axis_index("c")
        base = wid * _CL_PER_W
        himask = jnp.full((_LANES,), -65536, jnp.int32)
        nch = _CL_PER_W // _CHUNK

        def incp(ch, par):
            c0 = jnp.minimum(base + ch * _CHUNK, _LAST_START)
            return pltpu.make_async_copy(
                embT_hbm.at[:, pl.ds(c0, _CHUNK)], in_v.at[par],
                sem.at[par])

        incp(0, 0).start()

        def chunk(ch, carry):
            par = ch & 1

            @pl.when(ch < nch - 1)
            def _():
                incp(ch + 1, 1 - par).start()

            incp(ch, par).wait()

            def grp(g, carry2):
                rows = lax.iota(jnp.int32, _LANES) + g * _LANES
                for t in range(32):
                    a = plsc.bitcast(
                        in_v[par, 2 * t, pl.ds(g * _LANES, _LANES)],
                        jnp.int32)
                    b = plsc.bitcast(
                        in_v[par, 2 * t + 1, pl.ds(g * _LANES, _LANES)],
                        jnp.int32)
                    w = lax.bitwise_or(
                        lax.shift_right_logical(a, 16),
                        lax.bitwise_and(b, himask))
                    plsc.store_scatter(
                        out_v, [rows, jnp.full((_LANES,), t, jnp.int32)], w)
                return carry2

            lax.fori_loop(0, _CHUNK // _LANES, grp, 0)
            c0 = jnp.minimum(base + ch * _CHUNK, _LAST_START)
            pltpu.sync_copy(out_v.at[:, pl.ds(0, 32)],
                            out_hbm.at[pl.ds(c0, _CHUNK)])
            return carry

        lax.fori_loop(0, nch, chunk, 0)

    return k(embT)


def _sc_gather_sum(ids, packed):
    @functools.partial(
        pl.kernel,
        mesh=plsc.VectorSubcoreMesh(**_MESH),
        out_type=jax.ShapeDtypeStruct((_B, _DIM), jnp.float32),
        compiler_params=_CPARAMS,
        scratch_types=[
            pltpu.VMEM((_ROWS_PER_W, _L), jnp.int32),
            pltpu.VMEM((4, _L, 32), jnp.int32),
            pltpu.VMEM((_ROWS_PER_W, _DIM), jnp.float32),
            pltpu.SemaphoreType.DMA((4,)),
        ],
    )
    def k(ids_hbm, emb_hbm, out_hbm, idx_v, buf_v, acc_v, sem):
        wid = lax.axis_index("s") * _NC + lax.axis_index("c")
        base = wid * _ROWS_PER_W
        pltpu.sync_copy(ids_hbm.at[pl.ds(base, _ROWS_PER_W)], idx_v)

        def copies(r, par):
            return (
                pltpu.make_async_copy(
                    emb_hbm.at[idx_v.at[r, pl.ds(0, 128)]],
                    buf_v.at[par, pl.ds(0, 128)], sem.at[par]),
                pltpu.make_async_copy(
                    emb_hbm.at[idx_v.at[r, pl.ds(128, _L - 128)]],
                    buf_v.at[par, pl.ds(128, _L - 128)], sem.at[par]),
            )

        def fire(r, par):
            for cp in copies(r, par):
                cp.start()

        def drain(r, par):
            for cp in copies(r, par):
                cp.wait()

        for rr in range(3):
            fire(rr, rr)
        himask = jnp.full((_LANES,), -65536, jnp.int32)

        def row(r, carry):
            par = r & 3

            @pl.when(r < _ROWS_PER_W - 3)
            def _():
                fire(r + 3, (r + 3) & 3)

            drain(r, par)

            def red(j, accs):
                a0, a1, a2, a3 = accs
                for u in range(4):
                    for c in range(2):
                        x = buf_v[par, j * 4 + u,
                                  pl.ds(c * _LANES, _LANES)]
                        lo = plsc.bitcast(lax.shift_left(x, 16), jnp.float32)
                        hi = plsc.bitcast(lax.bitwise_and(x, himask),
                                          jnp.float32)
                        if c == 0:
                            a0 = a0 + lo
                            a1 = a1 + hi
                        else:
                            a2 = a2 + lo
                            a3 = a3 + hi
                return (a0, a1, a2, a3)

            zeros = tuple(
                jnp.zeros((_LANES,), jnp.float32) for _ in range(4))
            accs = lax.fori_loop(0, _L // 4, red, zeros)
            for c in range(4):
                acc_v[r, pl.ds(c * _LANES, _LANES)] = accs[c]
            return carry

        lax.fori_loop(0, _ROWS_PER_W, row, 0)
        pltpu.sync_copy(acc_v, out_hbm.at[pl.ds(base, _ROWS_PER_W)])

    return k(ids, packed)


def _tc_head(input_ids, emb_sum, Wp, b2d):
    def body(ids_ref, es_ref, w_ref, b_ref, out_ref):
        ids = ids_ref[...]
        cnt = jnp.sum((ids != _PAD_ID).astype(jnp.float32), axis=1,
                      keepdims=True)
        pooled = es_ref[...] / jnp.maximum(cnt, 1.0)
        out_ref[...] = (
            jnp.dot(pooled, w_ref[...], preferred_element_type=jnp.float32)
            + b_ref[...])

    return pl.pallas_call(
        body,
        out_shape=jax.ShapeDtypeStruct((_B, _NUM_LABELS), jnp.float32),
    )(input_ids, emb_sum, Wp, b2d)


def kernel(input_ids, embedding, W, b):
    ids = input_ids.astype(jnp.int32)
    embTp = jnp.pad(embedding.T, ((0, 0), (0, _COLS_PAD - _NUM_CLUSTERS - 1)))
    packed = _sc_pack_table(embTp)
    emb_sum = _sc_gather_sum(ids, packed)
    Wp = W[jnp.asarray(_PERM), :]
    return _tc_head(ids, emb_sum, Wp, b.reshape(1, _NUM_LABELS))


# speedup vs baseline: 1.2345x; 1.0077x over previous
import functools

import jax
import jax.numpy as jnp
import numpy as np
from jax import lax
from jax.experimental import pallas as pl
from jax.experimental.pallas import tpu as pltpu
from jax.experimental.pallas import tpu_sc as plsc

_NUM_CLUSTERS = 100000
_DIM = 64
_NUM_LABELS = 128
_PAD_ID = _NUM_CLUSTERS
_B = 4096
_L = 200

_NC = 2
_NS = 16
_NW = _NC * _NS
_ROWS_PER_W = _B // _NW
_LANES = 16

_VROWS = 100352
_WPR = 40
_CL_PER_W = _VROWS // _NW
_CHUNK = 448
_COLS_PAD = 100096
_LAST_START = _COLS_PAD - _CHUNK

_PERM = np.concatenate([
    np.arange(0, 32, 2), np.arange(1, 32, 2),
    np.arange(32, 64, 2), np.arange(33, 64, 2),
])

_MESH = dict(core_axis_name="c", subcore_axis_name="s")
_CPARAMS = pltpu.CompilerParams(
    use_tc_tiling_on_sc=False, needs_layout_passes=False)


def _sc_pack_table(embT):
    @functools.partial(
        pl.kernel,
        mesh=plsc.VectorSubcoreMesh(**_MESH),
        out_type=jax.ShapeDtypeStruct((_VROWS, 32), jnp.int32),
        compiler_params=_CPARAMS,
        scratch_types=[
            pltpu.VMEM((2, _DIM, _CHUNK), jnp.float32),
            pltpu.VMEM((_CHUNK, _WPR), jnp.int32),
            pltpu.SemaphoreType.DMA((2,)),
        ],
    )
    def k(embT_hbm, out_hbm, in_v, out_v, sem):
        wid = lax.axis_index("s") * _NC + lax.axis_index("c")
        base = wid * _CL_PER_W
        himask = jnp.full((_LANES,), -65536, jnp.int32)
        nch = _CL_PER_W // _CHUNK

        def incp(ch, par):
            c0 = jnp.minimum(base + ch * _CHUNK, _LAST_START)
            return pltpu.make_async_copy(
                embT_hbm.at[:, pl.ds(c0, _CHUNK)], in_v.at[par],
                sem.at[par])

        incp(0, 0).start()

        def chunk(ch, carry):
            par = ch & 1

            @pl.when(ch < nch - 1)
            def _():
                incp(ch + 1, 1 - par).start()

            incp(ch, par).wait()

            def grp(g, carry2):
                rows = lax.iota(jnp.int32, _LANES) + g * _LANES
                for t in range(32):
                    a = plsc.bitcast(
                        in_v[par, 2 * t, pl.ds(g * _LANES, _LANES)],
                        jnp.int32)
                    b = plsc.bitcast(
                        in_v[par, 2 * t + 1, pl.ds(g * _LANES, _LANES)],
                        jnp.int32)
                    w = lax.bitwise_or(
                        lax.shift_right_logical(a, 16),
                        lax.bitwise_and(b, himask))
                    plsc.store_scatter(
                        out_v, [rows, jnp.full((_LANES,), t, jnp.int32)], w)
                return carry2

            lax.fori_loop(0, _CHUNK // _LANES, grp, 0)
            c0 = jnp.minimum(base + ch * _CHUNK, _LAST_START)
            pltpu.sync_copy(out_v.at[:, pl.ds(0, 32)],
                            out_hbm.at[pl.ds(c0, _CHUNK)])
            return carry

        lax.fori_loop(0, nch, chunk, 0)

    return k(embT)


def _sc_gather_sum(ids, packed):
    @functools.partial(
        pl.kernel,
        mesh=plsc.VectorSubcoreMesh(**_MESH),
        out_type=jax.ShapeDtypeStruct((_B, _DIM), jnp.float32),
        compiler_params=_CPARAMS,
        scratch_types=[
            pltpu.VMEM((_ROWS_PER_W, _L), jnp.int32),
            pltpu.VMEM((8, _L, 32), jnp.int32),
            pltpu.VMEM((_ROWS_PER_W, _DIM), jnp.float32),
            pltpu.SemaphoreType.DMA((8,)),
        ],
    )
    def k(ids_hbm, emb_hbm, out_hbm, idx_v, buf_v, acc_v, sem):
        wid = lax.axis_index("s") * _NC + lax.axis_index("c")
        base = wid * _ROWS_PER_W
        pltpu.sync_copy(ids_hbm.at[pl.ds(base, _ROWS_PER_W)], idx_v)

        def copies(r, par):
            return (
                pltpu.make_async_copy(
                    emb_hbm.at[idx_v.at[r, pl.ds(0, 128)]],
                    buf_v.at[par, pl.ds(0, 128)], sem.at[par]),
                pltpu.make_async_copy(
                    emb_hbm.at[idx_v.at[r, pl.ds(128, _L - 128)]],
                    buf_v.at[par, pl.ds(128, _L - 128)], sem.at[par]),
            )

        def fire(r, par):
            for cp in copies(r, par):
                cp.start()

        def drain(r, par):
            for cp in copies(r, par):
                cp.wait()

        for rr in range(7):
            fire(rr, rr)
        himask = jnp.full((_LANES,), -65536, jnp.int32)

        def row(r, carry):
            par = r & 7

            @pl.when(r < _ROWS_PER_W - 7)
            def _():
                fire(r + 7, (r + 7) & 7)

            drain(r, par)

            def red(j, accs):
                a0, a1, a2, a3 = accs
                for u in range(4):
                    for c in range(2):
                        x = buf_v[par, j * 4 + u,
                                  pl.ds(c * _LANES, _LANES)]
                        lo = plsc.bitcast(lax.shift_left(x, 16), jnp.float32)
                        hi = plsc.bitcast(lax.bitwise_and(x, himask),
                                          jnp.float32)
                        if c == 0:
                            a0 = a0 + lo
                            a1 = a1 + hi
                        else:
                            a2 = a2 + lo
                            a3 = a3 + hi
                return (a0, a1, a2, a3)

            zeros = tuple(
                jnp.zeros((_LANES,), jnp.float32) for _ in range(4))
            accs = lax.fori_loop(0, _L // 4, red, zeros)
            for c in range(4):
                acc_v[r, pl.ds(c * _LANES, _LANES)] = accs[c]
            return carry

        lax.fori_loop(0, _ROWS_PER_W, row, 0)
        pltpu.sync_copy(acc_v, out_hbm.at[pl.ds(base, _ROWS_PER_W)])

    return k(ids, packed)


def _tc_head(input_ids, emb_sum, Wp, b2d):
    def body(ids_ref, es_ref, w_ref, b_ref, out_ref):
        ids = ids_ref[...]
        cnt = jnp.sum((ids != _PAD_ID).astype(jnp.float32), axis=1,
                      keepdims=True)
        pooled = es_ref[...] / jnp.maximum(cnt, 1.0)
        out_ref[...] = (
            jnp.dot(pooled, w_ref[...], preferred_element_type=jnp.float32)
            + b_ref[...])

    return pl.pallas_call(
        body,
        out_shape=jax.ShapeDtypeStruct((_B, _NUM_LABELS), jnp.float32),
    )(input_ids, emb_sum, Wp, b2d)


def kernel(input_ids, embedding, W, b):
    ids = input_ids.astype(jnp.int32)
    embTp = jnp.pad(embedding.T, ((0, 0), (0, _COLS_PAD - _NUM_CLUSTERS - 1)))
    packed = _sc_pack_table(embTp)
    emb_sum = _sc_gather_sum(ids, packed)
    Wp = W[jnp.asarray(_PERM), :]
    return _tc_head(ids, emb_sum, Wp, b.reshape(1, _NUM_LABELS))
